# Initial kernel scaffold; baseline (speedup 1.0000x reference)
import jax
import jax.numpy as jnp
from jax.experimental import pallas as pl


def kernel(tag_pred):
    raise NotImplementedError("write your pallas kernel here")


# speedup vs baseline: 8.8049x; 8.8049x over previous
import numpy as np
import jax
import jax.numpy as jnp
from jax.experimental import pallas as pl
from jax.experimental.pallas import tpu as pltpu

_THRESHOLD = 0.5
_MARGIN = 0.5
_MIN_PIXELS = 64


def _grid_np(h, w):
    if w >= h:
        xm = np.broadcast_to(
            np.linspace(0.0, w / h, w, dtype=np.float32).reshape(1, 1, w), (1, h, w))
        ym = np.broadcast_to(
            np.linspace(0.0, 1.0, h, dtype=np.float32).reshape(1, h, 1), (1, h, w))
    else:
        xm = np.broadcast_to(
            np.linspace(0.0, 1.0, w, dtype=np.float32).reshape(1, 1, w), (1, h, w))
        ym = np.broadcast_to(
            np.linspace(0.0, h / w, h, dtype=np.float32).reshape(1, h, 1), (1, h, w))
    return np.concatenate([xm, ym], axis=0).astype(np.float32)


def _cluster_kernel(x_ref, grid_ref, out_ref, scores_ref, sx_ref, sy_ref,
                    prop_ref, rowmax_ref):
    h, w = out_ref.shape[1], out_ref.shape[2]

    seed = x_ref[0, 4]
    mask = seed > _THRESHOLD
    seedm = 1.0 / (1.0 + jnp.exp(-seed))
    sx_ref[...] = jnp.where(mask, jnp.tanh(x_ref[0, 0]) + grid_ref[0],
                            jnp.float32(jnp.inf))
    sy_ref[...] = jnp.tanh(x_ref[0, 1]) + grid_ref[1]
    scores0 = jnp.where(mask, seedm, 0.0)
    scores_ref[...] = scores0
    rowmax_ref[...] = jnp.max(scores0, axis=1, keepdims=True)
    out_ref[0] = jnp.zeros((h, w), jnp.uint8)

    ucnt0 = jnp.sum(mask.astype(jnp.int32))
    rows_i = jax.lax.broadcasted_iota(jnp.int32, (h, 1), 0)
    cols_i = jax.lax.broadcasted_iota(jnp.int32, (1, w), 1)

    def cond(carry):
        ucnt, cnt, done, pend, plbl = carry
        return jnp.logical_and(ucnt > _MIN_PIXELS, done == 0)

    def body(carry):
        ucnt, cnt, done, pend, plbl = carry

        @pl.when(pend == 1)
        def _():
            out_ref[0] = jnp.where(prop_ref[...] != 0,
                                   plbl.astype(jnp.uint8), out_ref[0])

        rowmax = rowmax_ref[...]
        m = jnp.max(rowmax)
        r = jnp.min(jnp.where(rowmax == m, rows_i, h))
        srow = scores_ref[pl.ds(r, 1), :]
        c = jnp.min(jnp.where(srow == m, cols_i, w))
        new_done = (m < _THRESHOLD).astype(jnp.int32)
        active = new_done == 0

        cmask = cols_i == c
        cx = jnp.sum(jnp.where(cmask, sx_ref[pl.ds(r, 1), :], 0.0))
        cy = jnp.sum(jnp.where(cmask, sy_ref[pl.ds(r, 1), :], 0.0))
        s0 = jnp.sum(jnp.where(cmask, x_ref[0, 2, pl.ds(r, 1), :], 0.0))
        s1 = jnp.sum(jnp.where(cmask, x_ref[0, 3, pl.ds(r, 1), :], 0.0))

        s = scores_ref[...]
        d = ((sx_ref[...] - cx) ** 2 / (2.0 * s0 ** 2)
             + (sy_ref[...] - cy) ** 2 / (2.0 * s1 ** 2))
        dist = jnp.exp(-1.0 * d)
        proposal = dist > _MARGIN
        prop_ref[...] = proposal.astype(jnp.uint8)

        onehot = jnp.logical_and(rows_i == r, cols_i == c)
        psum = jnp.sum(proposal.astype(jnp.int32))
        pu = jnp.sum(jnp.logical_and(proposal, s > 0.0).astype(jnp.int32))
        pseed = jnp.sum(jnp.logical_and(proposal, onehot).astype(jnp.int32))
        usum = pu - pseed
        do_assign = jnp.logical_and(
            active,
            jnp.logical_and(psum > _MIN_PIXELS, 2 * usum > psum))
        new_s = jnp.where(
            jnp.logical_and(active, jnp.logical_or(proposal, onehot)), 0.0, s)
        scores_ref[...] = new_s
        rowmax_ref[...] = jnp.max(new_s, axis=1, keepdims=True)

        new_pend = jnp.where(do_assign, jnp.int32(1), jnp.int32(0))
        new_plbl = cnt % 256
        cnt = cnt + jnp.where(do_assign, 1, 0)
        ucnt = jnp.where(active, ucnt - 1 - usum, ucnt)
        return (ucnt, cnt, new_done, new_pend, new_plbl)

    ucnt_f, cnt_f, done_f, pend_f, plbl_f = jax.lax.while_loop(
        cond, body, (ucnt0, jnp.int32(1), jnp.int32(0), jnp.int32(0),
                     jnp.int32(0)))

    @pl.when(pend_f == 1)
    def _():
        out_ref[0] = jnp.where(prop_ref[...] != 0,
                               plbl_f.astype(jnp.uint8), out_ref[0])


def kernel(tag_pred):
    x = jnp.asarray(tag_pred, dtype=jnp.float32)
    b, ch, h, w = x.shape
    grid_c = jnp.asarray(_grid_np(h, w))
    return pl.pallas_call(
        _cluster_kernel,
        grid=(b,),
        in_specs=[
            pl.BlockSpec((1, ch, h, w), lambda i: (i, 0, 0, 0)),
            pl.BlockSpec((2, h, w), lambda i: (0, 0, 0)),
        ],
        out_specs=pl.BlockSpec((1, h, w), lambda i: (i, 0, 0)),
        out_shape=jax.ShapeDtypeStruct((b, h, w), jnp.uint8),
        scratch_shapes=[
            pltpu.VMEM((h, w), jnp.float32),
            pltpu.VMEM((h, w), jnp.float32),
            pltpu.VMEM((h, w), jnp.float32),
            pltpu.VMEM((h, w), jnp.uint8),
            pltpu.VMEM((h, 1), jnp.float32),
        ],
        compiler_params=pltpu.CompilerParams(
            dimension_semantics=("arbitrary",),
        ),
    )(x, grid_c)
